# baseline (speedup 1.0000x reference)
import jax
import jax.numpy as jnp
from jax.experimental import pallas as pl


def _body(x_ref, w_ref, b_ref, o_ref):
    x = x_ref[...].astype(jnp.bfloat16)
    y = jnp.einsum('ned,efd->nef', x, w_ref[...],
                   preferred_element_type=jnp.float32)
    o_ref[...] = y + b_ref[...]


def kernel(inputs, W, b):
    N, E, D = inputs.shape
    BN = 256
    w_bf = W.astype(jnp.bfloat16)
    return pl.pallas_call(
        _body,
        grid=(N // BN,),
        in_specs=[
            pl.BlockSpec((BN, E, D), lambda i: (i, 0, 0)),
            pl.BlockSpec((E, D, D), lambda i: (0, 0, 0)),
            pl.BlockSpec((E, D), lambda i: (0, 0)),
        ],
        out_specs=pl.BlockSpec((BN, E, D), lambda i: (i, 0, 0)),
        out_shape=jax.ShapeDtypeStruct((N, E, D), jnp.float32),
    )(inputs, w_bf, b)


# speedup vs baseline: 1.5590x; 1.5590x over previous
import jax
import jax.numpy as jnp
from jax.experimental import pallas as pl


def _body(x_ref, w_ref, b_ref, o_ref):
    y = jax.lax.dot_general(
        x_ref[...], w_ref[...],
        (((2,), (2,)), ((1,), (0,))),
        preferred_element_type=jnp.float32,
        precision=jax.lax.Precision.DEFAULT,
    )
    o_ref[...] = y.swapaxes(0, 1) + b_ref[...]


def kernel(inputs, W, b):
    N, E, D = inputs.shape
    BN = 256
    return pl.pallas_call(
        _body,
        grid=(N // BN,),
        in_specs=[
            pl.BlockSpec((BN, E, D), lambda i: (i, 0, 0)),
            pl.BlockSpec((E, D, D), lambda i: (0, 0, 0)),
            pl.BlockSpec((E, D), lambda i: (0, 0)),
        ],
        out_specs=pl.BlockSpec((BN, E, D), lambda i: (i, 0, 0)),
        out_shape=jax.ShapeDtypeStruct((N, E, D), jnp.float32),
    )(inputs, W, b)


# speedup vs baseline: 1.5650x; 1.0038x over previous
import jax
import jax.numpy as jnp
from jax.experimental import pallas as pl


def _body(x_ref, w_ref, b_ref, o_ref):
    x = x_ref[...].astype(jnp.bfloat16)
    y = jax.lax.dot_general(
        x, w_ref[...],
        (((2,), (2,)), ((1,), (0,))),
        preferred_element_type=jnp.float32,
    )
    o_ref[...] = y.swapaxes(0, 1) + b_ref[...]


def kernel(inputs, W, b):
    N, E, D = inputs.shape
    BN = 256
    w_bf = W.astype(jnp.bfloat16)
    return pl.pallas_call(
        _body,
        grid=(N // BN,),
        in_specs=[
            pl.BlockSpec((BN, E, D), lambda i: (i, 0, 0)),
            pl.BlockSpec((E, D, D), lambda i: (0, 0, 0)),
            pl.BlockSpec((E, D), lambda i: (0, 0)),
        ],
        out_specs=pl.BlockSpec((BN, E, D), lambda i: (i, 0, 0)),
        out_shape=jax.ShapeDtypeStruct((N, E, D), jnp.float32),
    )(inputs, w_bf, b)
